# Initial kernel scaffold; baseline (speedup 1.0000x reference)
import numpy as np
import jax
import jax.numpy as jnp
from jax import lax
from jax.experimental import pallas as pl
from jax.experimental.pallas import tpu as pltpu
from jax.experimental.pallas import tpu_sc as plsc

BATCH = 4096
ED = 32
BLK = 512
NBLK = BATCH // BLK
NPART = 50
NTAG = 20
EPS = 1e-5
NWORK = 32
NELEM = float(BATCH * ED)

_S32 = [('birthyear', 100), ('price', 1000), ('country', 200), ('city', 1000),
        ('maxprice', 1000), ('minprice', 1000), ('planner', 10000)]
_S32_OFF = np.concatenate([[0], np.cumsum([v for _, v in _S32])])
_TAGBAG_OFF = int(_S32_OFF[-1])
_S16 = [('gender', 3), ('level', 10), ('constellation', 12), ('province', 40),
        ('participant_num', 500)]
_S16_OFF = np.concatenate([[0], np.cumsum([v for _, v in _S16])])

_R_BIG = BATCH // NWORK
_R_S32 = len(_S32) * BATCH // NWORK
_R_S16 = len(_S16) * BATCH // NWORK
_R_PART = BATCH * NPART // NWORK
_R_TAG = BATCH * NTAG // NWORK
_CH = 1280

_FEATS = ['uid', 'gender', 'level', 'constellation', 'birthyear', 'region',
          'price', 'pid', 'owner', 'planner', 'maxprice', 'minprice',
          'participant_num', 'participants', 'project_tagid']
NF = len(_FEATS)


def _sc_gather_body(tab_uid, tab_pid, tab_owner, tab_parts, tab_s32, tab_s16,
                    i_uid, i_pid, i_owner, i_parts, i_s32, i_tag, i_s16,
                    o_uid, o_pid, o_owner, o_parts, o_s32, o_tag, o_s16,
                    idx128, idx896, idx640, idx1280,
                    r128, r896, r640, r1280, sem):
    wid = lax.axis_index("s") * 2 + lax.axis_index("c")

    def gath(tab, idx_hbm, out_hbm, base, n, idx_v, rows_v):
        pltpu.sync_copy(idx_hbm.at[pl.ds(base, n)], idx_v)
        pltpu.async_copy(tab.at[idx_v], rows_v, sem).wait()
        pltpu.sync_copy(rows_v, out_hbm.at[pl.ds(base, n)])

    gath(tab_uid, i_uid, o_uid, wid * _R_BIG, _R_BIG, idx128, r128)
    gath(tab_pid, i_pid, o_pid, wid * _R_BIG, _R_BIG, idx128, r128)
    gath(tab_owner, i_owner, o_owner, wid * _R_BIG, _R_BIG, idx128, r128)
    gath(tab_s32, i_s32, o_s32, wid * _R_S32, _R_S32, idx896, r896)
    gath(tab_s16, i_s16, o_s16, wid * _R_S16, _R_S16, idx640, r640)
    for k in range(_R_PART // _CH):
        gath(tab_parts, i_parts, o_parts, wid * _R_PART + k * _CH, _CH,
             idx1280, r1280)
    for k in range(_R_TAG // _CH):
        gath(tab_s32, i_tag, o_tag, wid * _R_TAG + k * _CH, _CH,
             idx1280, r1280)


def _sc_gather(tab_uid, tab_pid, tab_owner, tab_parts, tab_s32, tab_s16,
               i_uid, i_pid, i_owner, i_parts, i_s32, i_tag, i_s16):
    f32 = jnp.float32
    out_type = [
        jax.ShapeDtypeStruct((BATCH, 32), f32),
        jax.ShapeDtypeStruct((BATCH, 32), f32),
        jax.ShapeDtypeStruct((BATCH, 32), f32),
        jax.ShapeDtypeStruct((BATCH * NPART, 32), f32),
        jax.ShapeDtypeStruct((len(_S32) * BATCH, 32), f32),
        jax.ShapeDtypeStruct((BATCH * NTAG, 32), f32),
        jax.ShapeDtypeStruct((len(_S16) * BATCH, 16), f32),
    ]
    mesh = plsc.VectorSubcoreMesh(core_axis_name="c", subcore_axis_name="s")
    kern = pl.kernel(
        _sc_gather_body,
        out_type=out_type,
        mesh=mesh,
        scratch_types=[
            pltpu.VMEM((128,), jnp.int32),
            pltpu.VMEM((_R_S32,), jnp.int32),
            pltpu.VMEM((_R_S16,), jnp.int32),
            pltpu.VMEM((_CH,), jnp.int32),
            pltpu.VMEM((128, 32), f32),
            pltpu.VMEM((_R_S32, 32), f32),
            pltpu.VMEM((_R_S16, 16), f32),
            pltpu.VMEM((_CH, 32), f32),
            pltpu.SemaphoreType.DMA,
        ],
    )
    return kern(tab_uid, tab_pid, tab_owner, tab_parts, tab_s32, tab_s16,
                i_uid, i_pid, i_owner, i_parts, i_s32, i_tag, i_s16)


def _tc_a_body(*refs):
    (uid, gen, lev, con, by, pr, cty, prov, city,
     pid, own, plan, maxp, minp, pnum, parts, tag) = refs[:17]
    w = refs[17:-2]
    y_ref, st_ref = refs[-2], refs[-1]

    def lin(x_ref, i):
        return jnp.dot(x_ref[...], w[2 * i][...],
                       preferred_element_type=jnp.float32) + w[2 * i + 1][...]

    def linrelu(x_ref, i):
        return jnp.maximum(lin(x_ref, i), 0.0)

    y_uid = linrelu(uid, 0)
    y_gen = linrelu(gen, 1)
    y_lev = linrelu(lev, 2)
    y_con = linrelu(con, 3)
    y_by = linrelu(by, 4)
    y_pr = linrelu(pr, 5)
    reg = jnp.concatenate([lin(cty, 6), lin(prov, 7), lin(city, 8)], axis=1)
    y_reg = jnp.maximum(
        jnp.dot(reg, w[18][...], preferred_element_type=jnp.float32)
        + w[19][...], 0.0)
    y_pid = linrelu(pid, 10)
    y_own = linrelu(own, 11)
    y_plan = linrelu(plan, 12)
    y_maxp = linrelu(maxp, 13)
    y_minp = linrelu(minp, 14)
    y_pnum = linrelu(pnum, 15)
    y_parts = linrelu(parts, 16)
    y_tag = linrelu(tag, 17)

    y = jnp.concatenate(
        [y_uid, y_gen, y_lev, y_con, y_by, y_reg, y_pr,
         y_pid, y_own, y_plan, y_maxp, y_minp, y_pnum, y_parts, y_tag],
        axis=1)
    y_ref[...] = y
    s = jnp.sum(y, axis=0, keepdims=True)
    ss = jnp.sum(y * y, axis=0, keepdims=True)
    st = jnp.concatenate([s, ss], axis=0)
    i = pl.program_id(0)

    @pl.when(i == 0)
    def _():
        st_ref[...] = st

    @pl.when(i != 0)
    def _():
        st_ref[...] = st_ref[...] + st


def _tc_b_body(y_ref, st_ref, g_ref, gt_ref, ga_ref, be_ref,
               wu_ref, bu_ref, wp_ref, bp_ref, fu_ref, fp_ref, o_ref):
    f32 = jnp.float32
    s2 = jnp.dot(st_ref[...], g_ref[...], preferred_element_type=f32)
    m = s2[0:1, :] * (1.0 / NELEM)
    ex2 = s2[1:2, :] * (1.0 / NELEM)
    v = ex2 - m * m
    inv = lax.rsqrt(v + EPS)
    a480 = jnp.dot(inv, gt_ref[...], preferred_element_type=f32) * ga_ref[...]
    c480 = be_ref[...] - jnp.dot(m * inv, gt_ref[...],
                                 preferred_element_type=f32) * ga_ref[...]
    z = y_ref[...] * a480 + c480
    fu = jnp.tanh(jnp.dot(z, wu_ref[...], preferred_element_type=f32)
                  + bu_ref[...])
    fp = jnp.tanh(jnp.dot(z, wp_ref[...], preferred_element_type=f32)
                  + bp_ref[...])
    fu_ref[...] = fu
    fp_ref[...] = fp
    o_ref[...] = jnp.sum(fu * fp, axis=1, keepdims=True)


def _full(shape):
    return pl.BlockSpec(shape, lambda i: tuple(0 for _ in shape))


def _dense_forward(p, g_uid, g_pid, g_owner, parts2d, tag2d, g_s32, g_s16):
    f32 = jnp.float32

    def b2(name):
        return p['b_' + name].reshape(1, -1)

    emb_in = []
    emb_specs = []
    for arr in (g_uid, g_pid, g_owner):
        emb_in.append(arr)
        emb_specs.append(pl.BlockSpec((BLK, 32), lambda i: (i, 0)))
    s32_view = {name: j for j, (name, _) in enumerate(_S32)}
    for name in ('birthyear', 'price', 'country', 'city',
                 'maxprice', 'minprice', 'planner'):
        j = s32_view[name]
        emb_in.append(g_s32)
        emb_specs.append(
            pl.BlockSpec((BLK, 32), lambda i, j=j: (j * NBLK + i, 0)))
    s16_view = {name: j for j, (name, _) in enumerate(_S16)}
    for name in ('gender', 'level', 'constellation', 'province',
                 'participant_num'):
        j = s16_view[name]
        emb_in.append(g_s16)
        emb_specs.append(
            pl.BlockSpec((BLK, 16), lambda i, j=j: (j * NBLK + i, 0)))
    emb_in.append(parts2d)
    emb_specs.append(pl.BlockSpec((BLK, NPART * 32), lambda i: (i, 0)))
    emb_in.append(tag2d)
    emb_specs.append(pl.BlockSpec((BLK, NTAG * 32), lambda i: (i, 0)))

    order = {'uid': 0, 'pid': 1, 'owner': 2,
             'birthyear': 3, 'price': 4, 'country': 5, 'city': 6,
             'maxprice': 7, 'minprice': 8, 'planner': 9,
             'gender': 10, 'level': 11, 'constellation': 12, 'province': 13,
             'participant_num': 14, 'parts': 15, 'tag': 16}
    body_order = ['uid', 'gender', 'level', 'constellation', 'birthyear',
                  'price', 'country', 'province', 'city',
                  'pid', 'owner', 'planner', 'maxprice', 'minprice',
                  'participant_num', 'parts', 'tag']
    emb_in2 = [emb_in[order[n]] for n in body_order]
    emb_specs2 = [emb_specs[order[n]] for n in body_order]

    wparts = jnp.tile(p['W_participants'], (NPART, 1))
    wtag = jnp.tile(p['W_project_tagid'], (NTAG, 1))
    wnames = ['uid', 'gender', 'level', 'constellation', 'birthyear', 'price',
              'country', 'province', 'city', 'region', 'pid', 'owner',
              'planner', 'maxprice', 'minprice', 'participant_num']
    weights = []
    for n in wnames:
        weights.append(p['W_' + n])
        weights.append(b2(n))
    weights.append(wparts)
    weights.append(b2('participants'))
    weights.append(wtag)
    weights.append(b2('project_tagid'))
    w_specs = [_full(wa.shape) for wa in weights]

    y_all, stats = pl.pallas_call(
        _tc_a_body,
        grid=(NBLK,),
        in_specs=emb_specs2 + w_specs,
        out_specs=[pl.BlockSpec((BLK, NF * 32), lambda i: (i, 0)),
                   pl.BlockSpec((2, NF * 32), lambda i: (0, 0))],
        out_shape=[jax.ShapeDtypeStruct((BATCH, NF * 32), f32),
                   jax.ShapeDtypeStruct((2, NF * 32), f32)],
    )(*emb_in2, *weights)

    gmat = np.zeros((NF * 32, 16), np.float32)
    gmat[np.arange(NF * 32), np.arange(NF * 32) // 32] = 1.0
    gtmat = jnp.asarray(gmat.T.copy())
    gmat = jnp.asarray(gmat)
    ga480 = jnp.concatenate(
        [jnp.broadcast_to(p['g_' + f].reshape(1, 1), (1, 32)) for f in _FEATS],
        axis=1)
    be480 = jnp.concatenate(
        [jnp.broadcast_to(p['be_' + f].reshape(1, 1), (1, 32)) for f in _FEATS],
        axis=1)
    wu = jnp.concatenate([p['W_user_combine'],
                          jnp.zeros((8 * ED, 200), f32)], axis=0)
    wp = jnp.concatenate([jnp.zeros((7 * ED, 200), f32),
                          p['W_party_combine']], axis=0)
    bu = p['b_user_combine'].reshape(1, 200)
    bp = p['b_party_combine'].reshape(1, 200)

    fu, fp, out = pl.pallas_call(
        _tc_b_body,
        grid=(NBLK,),
        in_specs=[pl.BlockSpec((BLK, NF * 32), lambda i: (i, 0)),
                  _full((2, NF * 32)), _full((NF * 32, 16)),
                  _full((16, NF * 32)), _full((1, NF * 32)),
                  _full((1, NF * 32)), _full((NF * 32, 200)),
                  _full((1, 200)), _full((NF * 32, 200)), _full((1, 200))],
        out_specs=[pl.BlockSpec((BLK, 200), lambda i: (i, 0)),
                   pl.BlockSpec((BLK, 200), lambda i: (i, 0)),
                   pl.BlockSpec((BLK, 1), lambda i: (i, 0))],
        out_shape=[jax.ShapeDtypeStruct((BATCH, 200), f32),
                   jax.ShapeDtypeStruct((BATCH, 200), f32),
                   jax.ShapeDtypeStruct((BATCH, 1), f32)],
    )(y_all, stats, gmat, gtmat, ga480, be480, wu, bu, wp, bp)

    return (out, fu.reshape(BATCH, 1, 200), fp.reshape(BATCH, 1, 200))


def kernel(params, uid, gender, level, constellation, birthyear, country,
           province, city, price, pid, owner, planner, maxprice, minprice,
           participant_num, participants, project_tagid):
    p = params
    i32 = jnp.int32

    tab_s32 = jnp.concatenate(
        [p['E_' + n] for n, _ in _S32] + [p['Bag_project_tagid']], axis=0)
    tab_s16 = jnp.concatenate([p['E_' + n] for n, _ in _S16], axis=0)

    sing32 = {'birthyear': birthyear, 'price': price, 'country': country,
              'city': city, 'maxprice': maxprice, 'minprice': minprice,
              'planner': planner}
    i_s32 = jnp.concatenate(
        [sing32[n].reshape(-1).astype(i32) + int(_S32_OFF[j])
         for j, (n, _) in enumerate(_S32)])
    sing16 = {'gender': gender, 'level': level, 'constellation': constellation,
              'province': province, 'participant_num': participant_num}
    i_s16 = jnp.concatenate(
        [sing16[n].reshape(-1).astype(i32) + int(_S16_OFF[j])
         for j, (n, _) in enumerate(_S16)])
    i_tag = project_tagid.reshape(-1).astype(i32) + _TAGBAG_OFF

    g_uid, g_pid, g_owner, g_parts, g_s32, g_tag, g_s16 = _sc_gather(
        p['E_uid'], p['E_pid'], p['E_owner'], p['Bag_participants'],
        tab_s32, tab_s16,
        uid.reshape(-1).astype(i32), pid.reshape(-1).astype(i32),
        owner.reshape(-1).astype(i32), participants.reshape(-1).astype(i32),
        i_s32, i_tag, i_s16)

    parts2d = g_parts.reshape(BATCH, NPART * 32)
    tag2d = g_tag.reshape(BATCH, NTAG * 32)
    return _dense_forward(p, g_uid, g_pid, g_owner, parts2d, tag2d,
                          g_s32, g_s16)


# speedup vs baseline: 4.1428x; 4.1428x over previous
import numpy as np
import jax
import jax.numpy as jnp
from jax import lax
from jax.experimental import pallas as pl
from jax.experimental.pallas import tpu as pltpu
from jax.experimental.pallas import tpu_sc as plsc

BATCH = 4096
ED = 32
BLK = 512
NBLK = BATCH // BLK
NPART = 50
NTAG = 20
EPS = 1e-5
NWORK = 32
NELEM = float(BATCH * ED)

_S32 = [('birthyear', 100), ('price', 1000), ('country', 200), ('city', 1000),
        ('maxprice', 1000), ('minprice', 1000), ('planner', 10000)]
_S32_OFF = np.concatenate([[0], np.cumsum([v for _, v in _S32])])
_TAGBAG_OFF = int(_S32_OFF[-1])
_S16 = [('gender', 3), ('level', 10), ('constellation', 12), ('province', 40),
        ('participant_num', 500)]
_S16_OFF = np.concatenate([[0], np.cumsum([v for _, v in _S16])])

_R_BIG = BATCH // NWORK
_R_S32 = len(_S32) * BATCH // NWORK
_R_S16 = len(_S16) * BATCH // NWORK
_R_PART = BATCH * NPART // NWORK
_R_TAG = BATCH * NTAG // NWORK
_CH = 1280

_FEATS = ['uid', 'gender', 'level', 'constellation', 'birthyear', 'region',
          'price', 'pid', 'owner', 'planner', 'maxprice', 'minprice',
          'participant_num', 'participants', 'project_tagid']
NF = len(_FEATS)


def _sc_gather_body(tab_uid, tab_pid, tab_owner, tab_parts, tab_s32, tab_s16,
                    i_uid, i_pid, i_owner, i_parts, i_s32, i_tag, i_s16,
                    o_uid, o_pid, o_owner, o_parts, o_s32, o_tag, o_s16,
                    idx128, idx896, idx640, idx1280,
                    r128, r896, r640, r1280, sem):
    wid = lax.axis_index("s") * 2 + lax.axis_index("c")

    def gath(tab, idx_hbm, out_hbm, base, n, idx_v, rows_v):
        pltpu.sync_copy(idx_hbm.at[pl.ds(base, n)], idx_v)
        pltpu.async_copy(tab.at[idx_v], rows_v, sem).wait()
        pltpu.sync_copy(rows_v, out_hbm.at[pl.ds(base, n)])

    gath(tab_uid, i_uid, o_uid, wid * _R_BIG, _R_BIG, idx128, r128)
    gath(tab_pid, i_pid, o_pid, wid * _R_BIG, _R_BIG, idx128, r128)
    gath(tab_owner, i_owner, o_owner, wid * _R_BIG, _R_BIG, idx128, r128)
    gath(tab_s32, i_s32, o_s32, wid * _R_S32, _R_S32, idx896, r896)
    gath(tab_s16, i_s16, o_s16, wid * _R_S16, _R_S16, idx640, r640)
    for k in range(_R_PART // _CH):
        gath(tab_parts, i_parts, o_parts, wid * _R_PART + k * _CH, _CH,
             idx1280, r1280)
    for k in range(_R_TAG // _CH):
        gath(tab_s32, i_tag, o_tag, wid * _R_TAG + k * _CH, _CH,
             idx1280, r1280)


def _sc_gather(tab_uid, tab_pid, tab_owner, tab_parts, tab_s32, tab_s16,
               i_uid, i_pid, i_owner, i_parts, i_s32, i_tag, i_s16):
    f32 = jnp.float32
    out_type = [
        jax.ShapeDtypeStruct((BATCH, 32), f32),
        jax.ShapeDtypeStruct((BATCH, 32), f32),
        jax.ShapeDtypeStruct((BATCH, 32), f32),
        jax.ShapeDtypeStruct((BATCH * NPART, 32), f32),
        jax.ShapeDtypeStruct((len(_S32) * BATCH, 32), f32),
        jax.ShapeDtypeStruct((BATCH * NTAG, 32), f32),
        jax.ShapeDtypeStruct((len(_S16) * BATCH, 16), f32),
    ]
    mesh = plsc.VectorSubcoreMesh(core_axis_name="c", subcore_axis_name="s")
    kern = pl.kernel(
        _sc_gather_body,
        out_type=out_type,
        mesh=mesh,
        compiler_params=pltpu.CompilerParams(use_tc_tiling_on_sc=False),
        scratch_types=[
            pltpu.VMEM((128,), jnp.int32),
            pltpu.VMEM((_R_S32,), jnp.int32),
            pltpu.VMEM((_R_S16,), jnp.int32),
            pltpu.VMEM((_CH,), jnp.int32),
            pltpu.VMEM((128, 32), f32),
            pltpu.VMEM((_R_S32, 32), f32),
            pltpu.VMEM((_R_S16, 16), f32),
            pltpu.VMEM((_CH, 32), f32),
            pltpu.SemaphoreType.DMA,
        ],
    )
    return kern(tab_uid, tab_pid, tab_owner, tab_parts, tab_s32, tab_s16,
                i_uid, i_pid, i_owner, i_parts, i_s32, i_tag, i_s16)


def _tc_a_body(*refs):
    (uid, gen, lev, con, by, pr, cty, prov, city,
     pid, own, plan, maxp, minp, pnum, parts, tag) = refs[:17]
    w = refs[17:-2]
    y_ref, st_ref = refs[-2], refs[-1]

    def lin(x_ref, i):
        return jnp.dot(x_ref[...], w[2 * i][...],
                       preferred_element_type=jnp.float32) + w[2 * i + 1][...]

    def linrelu(x_ref, i):
        return jnp.maximum(lin(x_ref, i), 0.0)

    y_uid = linrelu(uid, 0)
    y_gen = linrelu(gen, 1)
    y_lev = linrelu(lev, 2)
    y_con = linrelu(con, 3)
    y_by = linrelu(by, 4)
    y_pr = linrelu(pr, 5)
    reg = jnp.concatenate([lin(cty, 6), lin(prov, 7), lin(city, 8)], axis=1)
    y_reg = jnp.maximum(
        jnp.dot(reg, w[18][...], preferred_element_type=jnp.float32)
        + w[19][...], 0.0)
    y_pid = linrelu(pid, 10)
    y_own = linrelu(own, 11)
    y_plan = linrelu(plan, 12)
    y_maxp = linrelu(maxp, 13)
    y_minp = linrelu(minp, 14)
    y_pnum = linrelu(pnum, 15)
    y_parts = linrelu(parts, 16)
    y_tag = linrelu(tag, 17)

    y = jnp.concatenate(
        [y_uid, y_gen, y_lev, y_con, y_by, y_reg, y_pr,
         y_pid, y_own, y_plan, y_maxp, y_minp, y_pnum, y_parts, y_tag],
        axis=1)
    y_ref[...] = y
    s = jnp.sum(y, axis=0, keepdims=True)
    ss = jnp.sum(y * y, axis=0, keepdims=True)
    st = jnp.concatenate([s, ss], axis=0)
    i = pl.program_id(0)

    @pl.when(i == 0)
    def _():
        st_ref[...] = st

    @pl.when(i != 0)
    def _():
        st_ref[...] = st_ref[...] + st


def _tc_b_body(y_ref, st_ref, g_ref, gt_ref, ga_ref, be_ref,
               wu_ref, bu_ref, wp_ref, bp_ref, fu_ref, fp_ref, o_ref):
    f32 = jnp.float32
    s2 = jnp.dot(st_ref[...], g_ref[...], preferred_element_type=f32)
    m = s2[0:1, :] * (1.0 / NELEM)
    ex2 = s2[1:2, :] * (1.0 / NELEM)
    v = ex2 - m * m
    inv = lax.rsqrt(v + EPS)
    a480 = jnp.dot(inv, gt_ref[...], preferred_element_type=f32) * ga_ref[...]
    c480 = be_ref[...] - jnp.dot(m * inv, gt_ref[...],
                                 preferred_element_type=f32) * ga_ref[...]
    z = y_ref[...] * a480 + c480
    fu = jnp.tanh(jnp.dot(z, wu_ref[...], preferred_element_type=f32)
                  + bu_ref[...])
    fp = jnp.tanh(jnp.dot(z, wp_ref[...], preferred_element_type=f32)
                  + bp_ref[...])
    fu_ref[...] = fu
    fp_ref[...] = fp
    o_ref[...] = jnp.sum(fu * fp, axis=1, keepdims=True)


def _full(shape):
    return pl.BlockSpec(shape, lambda i: tuple(0 for _ in shape))


def _dense_forward(p, g_uid, g_pid, g_owner, parts2d, tag2d, g_s32, g_s16):
    f32 = jnp.float32

    def b2(name):
        return p['b_' + name].reshape(1, -1)

    emb_in = []
    emb_specs = []
    for arr in (g_uid, g_pid, g_owner):
        emb_in.append(arr)
        emb_specs.append(pl.BlockSpec((BLK, 32), lambda i: (i, 0)))
    s32_view = {name: j for j, (name, _) in enumerate(_S32)}
    for name in ('birthyear', 'price', 'country', 'city',
                 'maxprice', 'minprice', 'planner'):
        j = s32_view[name]
        emb_in.append(g_s32)
        emb_specs.append(
            pl.BlockSpec((BLK, 32), lambda i, j=j: (j * NBLK + i, 0)))
    s16_view = {name: j for j, (name, _) in enumerate(_S16)}
    for name in ('gender', 'level', 'constellation', 'province',
                 'participant_num'):
        j = s16_view[name]
        emb_in.append(g_s16)
        emb_specs.append(
            pl.BlockSpec((BLK, 16), lambda i, j=j: (j * NBLK + i, 0)))
    emb_in.append(parts2d)
    emb_specs.append(pl.BlockSpec((BLK, NPART * 32), lambda i: (i, 0)))
    emb_in.append(tag2d)
    emb_specs.append(pl.BlockSpec((BLK, NTAG * 32), lambda i: (i, 0)))

    order = {'uid': 0, 'pid': 1, 'owner': 2,
             'birthyear': 3, 'price': 4, 'country': 5, 'city': 6,
             'maxprice': 7, 'minprice': 8, 'planner': 9,
             'gender': 10, 'level': 11, 'constellation': 12, 'province': 13,
             'participant_num': 14, 'parts': 15, 'tag': 16}
    body_order = ['uid', 'gender', 'level', 'constellation', 'birthyear',
                  'price', 'country', 'province', 'city',
                  'pid', 'owner', 'planner', 'maxprice', 'minprice',
                  'participant_num', 'parts', 'tag']
    emb_in2 = [emb_in[order[n]] for n in body_order]
    emb_specs2 = [emb_specs[order[n]] for n in body_order]

    wparts = jnp.tile(p['W_participants'], (NPART, 1))
    wtag = jnp.tile(p['W_project_tagid'], (NTAG, 1))
    wnames = ['uid', 'gender', 'level', 'constellation', 'birthyear', 'price',
              'country', 'province', 'city', 'region', 'pid', 'owner',
              'planner', 'maxprice', 'minprice', 'participant_num']
    weights = []
    for n in wnames:
        weights.append(p['W_' + n])
        weights.append(b2(n))
    weights.append(wparts)
    weights.append(b2('participants'))
    weights.append(wtag)
    weights.append(b2('project_tagid'))
    w_specs = [_full(wa.shape) for wa in weights]

    y_all, stats = pl.pallas_call(
        _tc_a_body,
        grid=(NBLK,),
        in_specs=emb_specs2 + w_specs,
        out_specs=[pl.BlockSpec((BLK, NF * 32), lambda i: (i, 0)),
                   pl.BlockSpec((2, NF * 32), lambda i: (0, 0))],
        out_shape=[jax.ShapeDtypeStruct((BATCH, NF * 32), f32),
                   jax.ShapeDtypeStruct((2, NF * 32), f32)],
    )(*emb_in2, *weights)

    gmat = np.zeros((NF * 32, 16), np.float32)
    gmat[np.arange(NF * 32), np.arange(NF * 32) // 32] = 1.0
    gtmat = jnp.asarray(gmat.T.copy())
    gmat = jnp.asarray(gmat)
    ga480 = jnp.concatenate(
        [jnp.broadcast_to(p['g_' + f].reshape(1, 1), (1, 32)) for f in _FEATS],
        axis=1)
    be480 = jnp.concatenate(
        [jnp.broadcast_to(p['be_' + f].reshape(1, 1), (1, 32)) for f in _FEATS],
        axis=1)
    wu = jnp.concatenate([p['W_user_combine'],
                          jnp.zeros((8 * ED, 200), f32)], axis=0)
    wp = jnp.concatenate([jnp.zeros((7 * ED, 200), f32),
                          p['W_party_combine']], axis=0)
    bu = p['b_user_combine'].reshape(1, 200)
    bp = p['b_party_combine'].reshape(1, 200)

    fu, fp, out = pl.pallas_call(
        _tc_b_body,
        grid=(NBLK,),
        in_specs=[pl.BlockSpec((BLK, NF * 32), lambda i: (i, 0)),
                  _full((2, NF * 32)), _full((NF * 32, 16)),
                  _full((16, NF * 32)), _full((1, NF * 32)),
                  _full((1, NF * 32)), _full((NF * 32, 200)),
                  _full((1, 200)), _full((NF * 32, 200)), _full((1, 200))],
        out_specs=[pl.BlockSpec((BLK, 200), lambda i: (i, 0)),
                   pl.BlockSpec((BLK, 200), lambda i: (i, 0)),
                   pl.BlockSpec((BLK, 1), lambda i: (i, 0))],
        out_shape=[jax.ShapeDtypeStruct((BATCH, 200), f32),
                   jax.ShapeDtypeStruct((BATCH, 200), f32),
                   jax.ShapeDtypeStruct((BATCH, 1), f32)],
    )(y_all, stats, gmat, gtmat, ga480, be480, wu, bu, wp, bp)

    return (out, fu.reshape(BATCH, 1, 200), fp.reshape(BATCH, 1, 200))


def kernel(params, uid, gender, level, constellation, birthyear, country,
           province, city, price, pid, owner, planner, maxprice, minprice,
           participant_num, participants, project_tagid):
    p = params
    i32 = jnp.int32

    tab_s32 = jnp.concatenate(
        [p['E_' + n] for n, _ in _S32] + [p['Bag_project_tagid']], axis=0)
    tab_s16 = jnp.concatenate([p['E_' + n] for n, _ in _S16], axis=0)

    sing32 = {'birthyear': birthyear, 'price': price, 'country': country,
              'city': city, 'maxprice': maxprice, 'minprice': minprice,
              'planner': planner}
    i_s32 = jnp.concatenate(
        [sing32[n].reshape(-1).astype(i32) + int(_S32_OFF[j])
         for j, (n, _) in enumerate(_S32)])
    sing16 = {'gender': gender, 'level': level, 'constellation': constellation,
              'province': province, 'participant_num': participant_num}
    i_s16 = jnp.concatenate(
        [sing16[n].reshape(-1).astype(i32) + int(_S16_OFF[j])
         for j, (n, _) in enumerate(_S16)])
    i_tag = project_tagid.reshape(-1).astype(i32) + _TAGBAG_OFF

    g_uid, g_pid, g_owner, g_parts, g_s32, g_tag, g_s16 = _sc_gather(
        p['E_uid'], p['E_pid'], p['E_owner'], p['Bag_participants'],
        tab_s32, tab_s16,
        uid.reshape(-1).astype(i32), pid.reshape(-1).astype(i32),
        owner.reshape(-1).astype(i32), participants.reshape(-1).astype(i32),
        i_s32, i_tag, i_s16)

    parts2d = g_parts.reshape(BATCH, NPART * 32)
    tag2d = g_tag.reshape(BATCH, NTAG * 32)
    return _dense_forward(p, g_uid, g_pid, g_owner, parts2d, tag2d,
                          g_s32, g_s16)
